# baseline (speedup 1.0000x reference)
import functools

import jax
import jax.numpy as jnp
from jax.experimental import pallas as pl
from jax.experimental.pallas import tpu as pltpu

_NEG_SLOPE = 0.2
_EPS = 1e-16
_LANES = 8


def _dense_kernel(x_ref, w_ref, ams_ref, amd_ref, h_ref, asrc_ref, adst_ref):
    h = jnp.dot(x_ref[...], w_ref[...], preferred_element_type=jnp.float32)
    h_ref[...] = h
    asrc_ref[...] = jnp.dot(h, ams_ref[...], preferred_element_type=jnp.float32)
    adst_ref[...] = jnp.dot(h, amd_ref[...], preferred_element_type=jnp.float32)


def _edge_softmax_kernel(src_ref, dst_ref, asrc_ref, adst_ref,
                         emax_ref, den_ref):
    phase = pl.program_id(0)
    blk = pl.program_id(1)
    eb = src_ref.shape[0]

    @pl.when((phase == 0) & (blk == 0))
    def _init():
        emax_ref[...] = jnp.full(emax_ref.shape, -jnp.inf, jnp.float32)
        den_ref[...] = jnp.zeros(den_ref.shape, jnp.float32)

    @pl.when(phase == 0)
    def _max_pass():
        def body(le, _):
            s = src_ref[le]
            d = dst_ref[le]
            ev = asrc_ref[pl.ds(s, 1), :] + adst_ref[pl.ds(d, 1), :]
            ev = jnp.where(ev >= 0, ev, _NEG_SLOPE * ev)
            emax_ref[pl.ds(d, 1), :] = jnp.maximum(
                emax_ref[pl.ds(d, 1), :], ev)
            return 0
        jax.lax.fori_loop(0, eb, body, 0)

    @pl.when(phase == 1)
    def _den_pass():
        def body(le, _):
            s = src_ref[le]
            d = dst_ref[le]
            ev = asrc_ref[pl.ds(s, 1), :] + adst_ref[pl.ds(d, 1), :]
            ev = jnp.where(ev >= 0, ev, _NEG_SLOPE * ev)
            exv = jnp.exp(ev - emax_ref[pl.ds(d, 1), :])
            den_ref[pl.ds(d, 1), :] = den_ref[pl.ds(d, 1), :] + exv
            return 0
        jax.lax.fori_loop(0, eb, body, 0)


def _message_kernel(src_ref, dst_ref, asrc_ref, pk_ref, h_ref, b_ref,
                    out_ref):
    head = pl.program_id(0)
    blk = pl.program_id(1)
    nblk = pl.num_programs(1)
    eb = src_ref.shape[0]
    hsel = (jax.lax.broadcasted_iota(jnp.int32, (1, _LANES), 1) ==
            head).astype(jnp.float32)

    @pl.when(blk == 0)
    def _init():
        out_ref[...] = jnp.zeros(out_ref.shape, jnp.float32)

    def body(le, _):
        s = src_ref[le]
        d = dst_ref[le]
        asr = asrc_ref[pl.ds(s, 1), :]
        pk = pk_ref[pl.ds(d, 1), :]
        ev = asr + pk[:, 0:_LANES]
        ev = jnp.where(ev >= 0, ev, _NEG_SLOPE * ev)
        av = jnp.exp(ev - pk[:, _LANES:2 * _LANES])
        av = av / (pk[:, 2 * _LANES:3 * _LANES] + _EPS)
        a = jnp.sum(av * hsel, axis=1, keepdims=True)
        out_ref[pl.ds(d, 1), :] = (out_ref[pl.ds(d, 1), :] +
                                   h_ref[pl.ds(s, 1), :] * a)
        return 0

    jax.lax.fori_loop(0, eb, body, 0)

    @pl.when(blk == nblk - 1)
    def _finish():
        v = out_ref[...] + b_ref[0]
        out_ref[...] = jnp.where(v > 0, v, jnp.exp(jnp.minimum(v, 0.0)) - 1.0)


def _pool_kernel(h_ref, batch_ref, wg1_ref, bg1_ref, wg2_ref, bg2_ref,
                 wf_ref, bf_ref, out_ref):
    h = h_ref[...]
    n, _ = h.shape
    g = out_ref.shape[0]
    hidden = jnp.dot(h, wg1_ref[...], preferred_element_type=jnp.float32)
    hidden = jnp.maximum(hidden + bg1_ref[...], 0.0)
    gate = jnp.dot(hidden, wg2_ref[...], preferred_element_type=jnp.float32)
    gate = gate + bg2_ref[...]

    gid = jax.lax.broadcasted_iota(jnp.int32, (n, g), 1)
    mask = gid == batch_ref[...]
    onehot = mask.astype(jnp.float32)
    masked = jnp.where(mask, gate, -jnp.inf)
    gmax = jnp.max(masked, axis=0, keepdims=True)
    gmax = jnp.where(jnp.isfinite(gmax), gmax, 0.0)
    gmax_n = jnp.sum(onehot * gmax, axis=1, keepdims=True)
    gexp = jnp.exp(gate - gmax_n)
    gden = jnp.sum(onehot * gexp, axis=0, keepdims=True)
    gden_n = jnp.sum(onehot * gden, axis=1, keepdims=True)
    a = gexp / (gden_n + _EPS)
    weighted = a * h
    pooled = jax.lax.dot_general(onehot, weighted, (((0,), (0,)), ((), ())),
                                 preferred_element_type=jnp.float32)
    out_ref[...] = jnp.dot(pooled, wf_ref[...],
                           preferred_element_type=jnp.float32) + bf_ref[...]


def _dense(x, w, ams, amd, row_block):
    n, in_ch = x.shape
    hh = w.shape[1]
    heads = ams.shape[1]
    grid = n // row_block
    return pl.pallas_call(
        _dense_kernel,
        grid=(grid,),
        in_specs=[
            pl.BlockSpec((row_block, in_ch), lambda i: (i, 0)),
            pl.BlockSpec((in_ch, hh), lambda i: (0, 0)),
            pl.BlockSpec((hh, heads), lambda i: (0, 0)),
            pl.BlockSpec((hh, heads), lambda i: (0, 0)),
        ],
        out_specs=[
            pl.BlockSpec((row_block, hh), lambda i: (i, 0)),
            pl.BlockSpec((row_block, heads), lambda i: (i, 0)),
            pl.BlockSpec((row_block, heads), lambda i: (i, 0)),
        ],
        out_shape=[
            jax.ShapeDtypeStruct((n, hh), jnp.float32),
            jax.ShapeDtypeStruct((n, heads), jnp.float32),
            jax.ShapeDtypeStruct((n, heads), jnp.float32),
        ],
    )(x, w, ams, amd)


def _edge_softmax(src, dst, asrc, adst):
    np_, lanes = asrc.shape
    e2p = src.shape[0]
    eb = min(4096, e2p)
    return pl.pallas_call(
        _edge_softmax_kernel,
        grid=(2, e2p // eb),
        in_specs=[
            pl.BlockSpec((eb,), lambda p, i: (i,), memory_space=pltpu.SMEM),
            pl.BlockSpec((eb,), lambda p, i: (i,), memory_space=pltpu.SMEM),
            pl.BlockSpec((np_, lanes), lambda p, i: (0, 0)),
            pl.BlockSpec((np_, lanes), lambda p, i: (0, 0)),
        ],
        out_specs=[
            pl.BlockSpec((np_, lanes), lambda p, i: (0, 0)),
            pl.BlockSpec((np_, lanes), lambda p, i: (0, 0)),
        ],
        out_shape=[
            jax.ShapeDtypeStruct((np_, lanes), jnp.float32),
            jax.ShapeDtypeStruct((np_, lanes), jnp.float32),
        ],
    )(src, dst, asrc, adst)


def _message(src, dst, asrc, packed, h, b, heads, ch):
    np_ = h.shape[0]
    e2p = src.shape[0]
    eb = min(4096, e2p)
    b3 = b.reshape(heads, 1, ch)
    return pl.pallas_call(
        _message_kernel,
        grid=(heads, e2p // eb),
        in_specs=[
            pl.BlockSpec((eb,), lambda hh, i: (i,), memory_space=pltpu.SMEM),
            pl.BlockSpec((eb,), lambda hh, i: (i,), memory_space=pltpu.SMEM),
            pl.BlockSpec((np_, _LANES), lambda hh, i: (0, 0)),
            pl.BlockSpec((np_, 3 * _LANES), lambda hh, i: (0, 0)),
            pl.BlockSpec((np_, ch), lambda hh, i: (0, hh)),
            pl.BlockSpec((1, 1, ch), lambda hh, i: (hh, 0, 0)),
        ],
        out_specs=pl.BlockSpec((np_, ch), lambda hh, i: (0, hh)),
        out_shape=jax.ShapeDtypeStruct((np_, heads * ch), jnp.float32),
    )(src, dst, asrc, packed, h, b3)


def _pool(h, batch, wg1, bg1, wg2, bg2, wf, bf, num_graphs):
    n, ch = h.shape
    nc = wf.shape[1]
    return pl.pallas_call(
        _pool_kernel,
        out_shape=jax.ShapeDtypeStruct((num_graphs, nc), jnp.float32),
    )(h, batch.reshape(n, 1).astype(jnp.int32), wg1, bg1.reshape(1, ch),
      wg2, bg2.reshape(1, 1), wf, bf.reshape(1, nc))


def _alpha_mat(a):
    heads, ch = a.shape
    eye = jnp.eye(heads, dtype=jnp.float32)
    return (a[:, :, None] * eye[:, None, :]).reshape(heads * ch, heads)


def _gat_layer(src, dst, x, w, a_src, a_dst, b, row_block):
    n = x.shape[0]
    heads, hid = a_src.shape
    h, asrc, adst = _dense(x, w, _alpha_mat(a_src), _alpha_mat(a_dst),
                           row_block)

    def padp(a):
        return jnp.pad(a, ((0, 8), (0, _LANES - a.shape[1])))

    asrc_p = padp(asrc)
    adst_p = padp(adst)
    emax, den = _edge_softmax(src, dst, asrc_p, adst_p)
    packed = jnp.concatenate([adst_p, emax, den], axis=1)
    hp = jnp.pad(h, ((0, 8), (0, 0)))
    return _message(src, dst, asrc_p, packed, hp, b, heads, hid)[:n]


def kernel(x, edge_index, batch, W1, a_src1, a_dst1, b1, W2, a_src2, a_dst2,
           b2, Wg1, bg1, Wg2, bg2, Wf, bf):
    n = x.shape[0]
    num_graphs = 64
    loop = jnp.arange(n, dtype=edge_index.dtype)
    src = jnp.concatenate([edge_index[0], loop]).astype(jnp.int32)
    dst = jnp.concatenate([edge_index[1], loop]).astype(jnp.int32)

    e2 = src.shape[0]
    e2p = ((e2 + 4095) // 4096) * 4096
    src = jnp.pad(src, (0, e2p - e2), constant_values=n)
    dst = jnp.pad(dst, (0, e2p - e2), constant_values=n)

    row_block = 400 if n % 400 == 0 else n

    h1 = _gat_layer(src, dst, x, W1, a_src1, a_dst1, b1, row_block)
    h2 = _gat_layer(src, dst, h1, W2, a_src2, a_dst2, b2, row_block)
    return _pool(h2, batch, Wg1, bg1, Wg2, bg2, Wf, bf, num_graphs)
